# baseline (speedup 1.0000x reference)
import functools

import jax
import jax.numpy as jnp
from jax.experimental import pallas as pl
from jax.experimental.pallas import tpu as pltpu

HIDDEN = 2048
C = 64
NH = 16
TOPK = 64
M = 16
B = 2
T = 4096
NB = T // M

TQ = 512
NT = T // TQ
NEG_INF = float("-inf")


def _compressor_body(h_ref, w_kv_ref, w_z_ref, bias_ref, kic_ref):
    hb = h_ref[0]
    dn = (((1,), (1,)), ((), ()))
    cb = jax.lax.dot_general(hb, w_kv_ref[...], dn,
                             preferred_element_type=jnp.float32)
    zb = jax.lax.dot_general(hb, w_z_ref[...], dn,
                             preferred_element_type=jnp.float32)
    nblk = TQ // M
    z = zb.reshape(nblk, M, C) + bias_ref[...][None, :, :]
    z = z - jnp.max(z, axis=1, keepdims=True)
    ez = jnp.exp(z)
    wgt = ez / jnp.sum(ez, axis=1, keepdims=True)
    kic = jnp.sum(wgt * cb.reshape(nblk, M, C), axis=1)
    kic_ref[0] = kic


def _scores_topk_body(h_ref, w_dq_ref, w_iuq_ref, w_w_ref, kic_ref, idx_ref):
    i = pl.program_id(1)
    hb = h_ref[0]
    dn = (((1,), (1,)), ((), ()))
    cq = jax.lax.dot_general(hb, w_dq_ref[...], dn,
                             preferred_element_type=jnp.float32)
    q = jax.lax.dot_general(cq, w_iuq_ref[...], dn,
                            preferred_element_type=jnp.float32)
    wv = jax.lax.dot_general(hb, w_w_ref[...], dn,
                             preferred_element_type=jnp.float32)
    kic = kic_ref[0]

    scores = jnp.zeros((TQ, NB), jnp.float32)
    for hd in range(NH):
        qh = q[:, hd * C:(hd + 1) * C]
        s = jax.lax.dot_general(qh, kic, dn,
                                preferred_element_type=jnp.float32)
        scores = scores + jnp.maximum(s, 0.0) * wv[:, hd:hd + 1]

    tvec = i * TQ + jax.lax.broadcasted_iota(jnp.int32, (TQ, NB), 0)
    bend = jax.lax.broadcasted_iota(jnp.int32, (TQ, NB), 1) * M + (M - 1)
    scores = jnp.where(bend < tvec, scores, NEG_INF)

    lane_nb = jax.lax.broadcasted_iota(jnp.int32, (TQ, NB), 1)
    lane_k = jax.lax.broadcasted_iota(jnp.int32, (TQ, TOPK), 1)

    def body(k, carry):
        sc, out = carry
        m = jnp.max(sc, axis=1, keepdims=True)
        hit = sc == m
        bi = jnp.min(jnp.where(hit, lane_nb, NB), axis=1,
                     keepdims=True)
        rec = jnp.where(m > NEG_INF, bi, -1)
        out = jnp.where(lane_k == k, rec, out)
        sc = jnp.where(lane_nb == bi, NEG_INF, sc)
        return sc, out

    out0 = jnp.full((TQ, TOPK), -1, jnp.int32)
    _, out = jax.lax.fori_loop(0, TOPK, body, (scores, out0))
    idx_ref[0] = out


@jax.jit
def kernel(h, w_a_kv, w_b_kv, w_a_z, w_b_z, bias_a, bias_b, w_dq, w_iuq, w_w):
    del w_a_kv, w_a_z, bias_a

    kic = pl.pallas_call(
        _compressor_body,
        grid=(B, NT),
        in_specs=[
            pl.BlockSpec((1, TQ, HIDDEN), lambda b, i: (b, i, 0)),
            pl.BlockSpec((C, HIDDEN), lambda b, i: (0, 0)),
            pl.BlockSpec((C, HIDDEN), lambda b, i: (0, 0)),
            pl.BlockSpec((M, C), lambda b, i: (0, 0)),
        ],
        out_specs=pl.BlockSpec((1, TQ // M, C), lambda b, i: (b, i, 0)),
        out_shape=jax.ShapeDtypeStruct((B, NB, C), jnp.float32),
    )(h, w_b_kv, w_b_z, bias_b)

    top_idx = pl.pallas_call(
        _scores_topk_body,
        grid=(B, NT),
        in_specs=[
            pl.BlockSpec((1, TQ, HIDDEN), lambda b, i: (b, i, 0)),
            pl.BlockSpec((C, HIDDEN), lambda b, i: (0, 0)),
            pl.BlockSpec((NH * C, C), lambda b, i: (0, 0)),
            pl.BlockSpec((NH, HIDDEN), lambda b, i: (0, 0)),
            pl.BlockSpec((1, NB, C), lambda b, i: (b, 0, 0)),
        ],
        out_specs=pl.BlockSpec((1, TQ, TOPK), lambda b, i: (b, i, 0)),
        out_shape=jax.ShapeDtypeStruct((B, T, TOPK), jnp.int32),
    )(h, w_dq, w_iuq, w_w, kic)

    return (kic, top_idx)


# speedup vs baseline: 1.0956x; 1.0956x over previous
import functools

import jax
import jax.numpy as jnp
from jax.experimental import pallas as pl
from jax.experimental.pallas import tpu as pltpu

HIDDEN = 2048
C = 64
NH = 16
TOPK = 64
M = 16
B = 2
T = 4096
NB = T // M

TQ = 512
NT = T // TQ
NEG_INF = float("-inf")


def _compressor_body(h_ref, w_kv_ref, w_z_ref, bias_ref, kic_ref):
    hb = h_ref[0]
    dn = (((1,), (1,)), ((), ()))
    cb = jax.lax.dot_general(hb, w_kv_ref[...], dn,
                             preferred_element_type=jnp.float32)
    zb = jax.lax.dot_general(hb, w_z_ref[...], dn,
                             preferred_element_type=jnp.float32)
    nblk = TQ // M
    z = zb.reshape(nblk, M, C) + bias_ref[...][None, :, :]
    z = z - jnp.max(z, axis=1, keepdims=True)
    ez = jnp.exp(z)
    wgt = ez / jnp.sum(ez, axis=1, keepdims=True)
    kic = jnp.sum(wgt * cb.reshape(nblk, M, C), axis=1)
    kic_ref[0] = kic


def _scores_topk_body(h_ref, w_dq_ref, w_iuq_ref, w_w_ref, kic_ref, idx_ref):
    i = pl.program_id(1)
    hb = h_ref[0]
    dn = (((1,), (1,)), ((), ()))
    cq = jax.lax.dot_general(hb, w_dq_ref[...], dn,
                             preferred_element_type=jnp.float32)
    q = jax.lax.dot_general(cq, w_iuq_ref[...], dn,
                            preferred_element_type=jnp.float32)
    wv = jax.lax.dot_general(hb, w_w_ref[...], dn,
                             preferred_element_type=jnp.float32)
    kic = kic_ref[0]

    scores = jnp.zeros((TQ, NB), jnp.float32)
    for hd in range(NH):
        qh = q[:, hd * C:(hd + 1) * C]
        s = jax.lax.dot_general(qh, kic, dn,
                                preferred_element_type=jnp.float32)
        scores = scores + jnp.maximum(s, 0.0) * wv[:, hd:hd + 1]

    tvec = i * TQ + jax.lax.broadcasted_iota(jnp.int32, (TQ, NB), 0)
    bend = jax.lax.broadcasted_iota(jnp.int32, (TQ, NB), 1) * M + (M - 1)
    scores = jnp.where(bend < tvec, scores, NEG_INF)

    lane = jax.lax.broadcasted_iota(jnp.int32, (TQ, NB), 1)
    v = scores
    ix = lane
    k = 2
    while k <= NB:
        d = k // 2
        while d >= 1:
            bit_d = (lane & d) != 0
            flip = bit_d == ((lane & k) == 0)
            pv = jnp.where(bit_d, pltpu.roll(v, d, 1),
                           pltpu.roll(v, NB - d, 1))
            pi = jnp.where(bit_d, pltpu.roll(ix, d, 1),
                           pltpu.roll(ix, NB - d, 1))
            sf = (v > pv) | ((v == pv) & (ix < pi))
            take = sf != flip
            v = jnp.where(take, v, pv)
            ix = jnp.where(take, ix, pi)
            d //= 2
        k *= 2
    idx_ref[0] = jnp.where(v[:, :TOPK] > NEG_INF, ix[:, :TOPK], -1)


@jax.jit
def kernel(h, w_a_kv, w_b_kv, w_a_z, w_b_z, bias_a, bias_b, w_dq, w_iuq, w_w):
    del w_a_kv, w_a_z, bias_a

    kic = pl.pallas_call(
        _compressor_body,
        grid=(B, NT),
        in_specs=[
            pl.BlockSpec((1, TQ, HIDDEN), lambda b, i: (b, i, 0)),
            pl.BlockSpec((C, HIDDEN), lambda b, i: (0, 0)),
            pl.BlockSpec((C, HIDDEN), lambda b, i: (0, 0)),
            pl.BlockSpec((M, C), lambda b, i: (0, 0)),
        ],
        out_specs=pl.BlockSpec((1, TQ // M, C), lambda b, i: (b, i, 0)),
        out_shape=jax.ShapeDtypeStruct((B, NB, C), jnp.float32),
    )(h, w_b_kv, w_b_z, bias_b)

    top_idx = pl.pallas_call(
        _scores_topk_body,
        grid=(B, NT),
        in_specs=[
            pl.BlockSpec((1, TQ, HIDDEN), lambda b, i: (b, i, 0)),
            pl.BlockSpec((C, HIDDEN), lambda b, i: (0, 0)),
            pl.BlockSpec((NH * C, C), lambda b, i: (0, 0)),
            pl.BlockSpec((NH, HIDDEN), lambda b, i: (0, 0)),
            pl.BlockSpec((1, NB, C), lambda b, i: (b, 0, 0)),
        ],
        out_specs=pl.BlockSpec((1, TQ, TOPK), lambda b, i: (b, i, 0)),
        out_shape=jax.ShapeDtypeStruct((B, T, TOPK), jnp.int32),
    )(h, w_dq, w_iuq, w_w, kic)

    return (kic, top_idx)


# speedup vs baseline: 1.3332x; 1.2168x over previous
import functools

import jax
import jax.numpy as jnp
from jax.experimental import pallas as pl
from jax.experimental.pallas import tpu as pltpu

HIDDEN = 2048
C = 64
NH = 16
TOPK = 64
M = 16
B = 2
T = 4096
NB = T // M

TQ = 512
NT = T // TQ
NEG_INF = float("-inf")


def _compressor_body(h_ref, w_kv_ref, w_z_ref, bias_ref, kic_ref):
    hb = h_ref[0]
    dn = (((1,), (1,)), ((), ()))
    cb = jax.lax.dot_general(hb, w_kv_ref[...], dn,
                             preferred_element_type=jnp.float32)
    zb = jax.lax.dot_general(hb, w_z_ref[...], dn,
                             preferred_element_type=jnp.float32)
    nblk = TQ // M
    z = zb.reshape(nblk, M, C) + bias_ref[...][None, :, :]
    z = z - jnp.max(z, axis=1, keepdims=True)
    ez = jnp.exp(z)
    wgt = ez / jnp.sum(ez, axis=1, keepdims=True)
    kic = jnp.sum(wgt * cb.reshape(nblk, M, C), axis=1)
    kic_ref[0] = kic


def _scores_topk_body(h_ref, w_dq_ref, w_iuq_ref, w_w_ref, kic_ref, idx_ref):
    i = pl.program_id(1)
    hb = h_ref[0]
    dn = (((1,), (1,)), ((), ()))
    cq = jax.lax.dot_general(hb, w_dq_ref[...], dn,
                             preferred_element_type=jnp.float32)
    q = jax.lax.dot_general(cq, w_iuq_ref[...], dn,
                            preferred_element_type=jnp.float32)
    wv = jax.lax.dot_general(hb, w_w_ref[...], dn,
                             preferred_element_type=jnp.float32)
    kic = kic_ref[0]

    scores = jnp.zeros((TQ, NB), jnp.float32)
    for hd in range(NH):
        qh = q[:, hd * C:(hd + 1) * C]
        s = jax.lax.dot_general(qh, kic, dn,
                                preferred_element_type=jnp.float32)
        scores = scores + jnp.maximum(s, 0.0) * wv[:, hd:hd + 1]

    tvec = i * TQ + jax.lax.broadcasted_iota(jnp.int32, (TQ, NB), 0)
    bend = jax.lax.broadcasted_iota(jnp.int32, (TQ, NB), 1) * M + (M - 1)
    scores = jnp.where(bend < tvec, scores, NEG_INF)

    def bstep(v, ix, lane, width, k, d):
        bit_d = (lane & d) != 0
        flip = bit_d == ((lane & k) == 0)
        pv = jnp.where(bit_d, pltpu.roll(v, d, 1),
                       pltpu.roll(v, width - d, 1))
        pi = jnp.where(bit_d, pltpu.roll(ix, d, 1),
                       pltpu.roll(ix, width - d, 1))
        sf = (v > pv) | ((v == pv) & (ix < pi))
        take = sf != flip
        return jnp.where(take, v, pv), jnp.where(take, ix, pi)

    lane = jax.lax.broadcasted_iota(jnp.int32, (TQ, NB), 1)
    v = scores
    ix = lane
    k = 2
    while k <= 64:
        d = k // 2
        while d >= 1:
            v, ix = bstep(v, ix, lane, NB, k, d)
            d //= 2
        k *= 2
    pv = pltpu.roll(v, NB - 64, 1)
    pi = pltpu.roll(ix, NB - 64, 1)
    sf = (v > pv) | ((v == pv) & (ix < pi))
    vm = jnp.where(sf, v, pv)
    im = jnp.where(sf, ix, pi)
    vs = pltpu.roll(vm, NB - 64, 1)
    is_ = pltpu.roll(im, NB - 64, 1)
    low = lane < 64
    v2 = jnp.where(low, vm, vs)[:, :128]
    i2 = jnp.where(low, im, is_)[:, :128]
    lane2 = jax.lax.broadcasted_iota(jnp.int32, (TQ, 128), 1)
    for d in (32, 16, 8, 4, 2, 1):
        v2, i2 = bstep(v2, i2, lane2, 128, 64, d)
    for d in (64, 32, 16, 8, 4, 2, 1):
        v2, i2 = bstep(v2, i2, lane2, 128, 128, d)
    idx_ref[0] = jnp.where(v2[:, :TOPK] > NEG_INF, i2[:, :TOPK], -1)


@jax.jit
def kernel(h, w_a_kv, w_b_kv, w_a_z, w_b_z, bias_a, bias_b, w_dq, w_iuq, w_w):
    del w_a_kv, w_a_z, bias_a

    kic = pl.pallas_call(
        _compressor_body,
        grid=(B, NT),
        in_specs=[
            pl.BlockSpec((1, TQ, HIDDEN), lambda b, i: (b, i, 0)),
            pl.BlockSpec((C, HIDDEN), lambda b, i: (0, 0)),
            pl.BlockSpec((C, HIDDEN), lambda b, i: (0, 0)),
            pl.BlockSpec((M, C), lambda b, i: (0, 0)),
        ],
        out_specs=pl.BlockSpec((1, TQ // M, C), lambda b, i: (b, i, 0)),
        out_shape=jax.ShapeDtypeStruct((B, NB, C), jnp.float32),
    )(h, w_b_kv, w_b_z, bias_b)

    top_idx = pl.pallas_call(
        _scores_topk_body,
        grid=(B, NT),
        in_specs=[
            pl.BlockSpec((1, TQ, HIDDEN), lambda b, i: (b, i, 0)),
            pl.BlockSpec((C, HIDDEN), lambda b, i: (0, 0)),
            pl.BlockSpec((NH * C, C), lambda b, i: (0, 0)),
            pl.BlockSpec((NH, HIDDEN), lambda b, i: (0, 0)),
            pl.BlockSpec((1, NB, C), lambda b, i: (b, 0, 0)),
        ],
        out_specs=pl.BlockSpec((1, TQ, TOPK), lambda b, i: (b, i, 0)),
        out_shape=jax.ShapeDtypeStruct((B, T, TOPK), jnp.int32),
    )(h, w_dq, w_iuq, w_w, kic)

    return (kic, top_idx)


# speedup vs baseline: 1.4057x; 1.0544x over previous
import jax
import jax.numpy as jnp
from jax.experimental import pallas as pl
from jax.experimental.pallas import tpu as pltpu

HIDDEN = 2048
C = 64
NH = 16
TOPK = 64
M = 16
B = 2
T = 4096
NB = T // M

TQ = 512
NT = T // TQ
NEG_INF = float("-inf")


def _body(h_ref, w_kv_ref, w_z_ref, bias_ref, w_dq_ref, w_iuq_ref, w_w_ref,
          kic_ref, idx_ref, kic_acc):
    i = pl.program_id(1)
    hb = h_ref[0]
    dn = (((1,), (1,)), ((), ()))

    cb = jax.lax.dot_general(hb, w_kv_ref[...], dn,
                             preferred_element_type=jnp.float32)
    zb = jax.lax.dot_general(hb, w_z_ref[...], dn,
                             preferred_element_type=jnp.float32)
    nblk = TQ // M
    z = zb.reshape(nblk, M, C) + bias_ref[...][None, :, :]
    z = z - jnp.max(z, axis=1, keepdims=True)
    ez = jnp.exp(z)
    wgt = ez / jnp.sum(ez, axis=1, keepdims=True)
    kic = jnp.sum(wgt * cb.reshape(nblk, M, C), axis=1)
    kic_ref[0] = kic
    kic_acc[pl.ds(i * nblk, nblk), :] = kic

    cq = jax.lax.dot_general(hb, w_dq_ref[...], dn,
                             preferred_element_type=jnp.float32)
    q = jax.lax.dot_general(cq, w_iuq_ref[...], dn,
                            preferred_element_type=jnp.float32)
    wv = jax.lax.dot_general(hb, w_w_ref[...], dn,
                             preferred_element_type=jnp.float32)
    kic_all = kic_acc[...]

    scores = jnp.zeros((TQ, NB), jnp.float32)
    for hd in range(NH):
        qh = q[:, hd * C:(hd + 1) * C]
        s = jax.lax.dot_general(qh, kic_all, dn,
                                preferred_element_type=jnp.float32)
        scores = scores + jnp.maximum(s, 0.0) * wv[:, hd:hd + 1]

    tvec = i * TQ + jax.lax.broadcasted_iota(jnp.int32, (TQ, NB), 0)
    bend = jax.lax.broadcasted_iota(jnp.int32, (TQ, NB), 1) * M + (M - 1)
    scores = jnp.where(bend < tvec, scores, NEG_INF)

    lane = jax.lax.broadcasted_iota(jnp.int32, (TQ, NB), 1)
    bitm = {d: (lane & d) != 0 for d in (1, 2, 4, 8, 16, 32, 64, 128)}
    km = {k: (lane & k) == 0 for k in (2, 4, 8, 16, 32, 64, 128)}
    lane2 = lane[:, :128]
    bitm2 = {d: bitm[d][:, :128] for d in (1, 2, 4, 8, 16, 32, 64)}
    km2 = {k: km[k][:, :128] for k in (64, 128)}

    def bstep(v, ix, width, bit_d, flip, d):
        pv = jnp.where(bit_d, pltpu.roll(v, d, 1),
                       pltpu.roll(v, width - d, 1))
        pi = jnp.where(bit_d, pltpu.roll(ix, d, 1),
                       pltpu.roll(ix, width - d, 1))
        sf = (v > pv) | ((v == pv) & (ix < pi))
        take = sf != flip
        return jnp.where(take, v, pv), jnp.where(take, ix, pi)

    v = scores
    ix = lane
    k = 2
    while k <= 64:
        d = k // 2
        while d >= 1:
            v, ix = bstep(v, ix, NB, bitm[d], bitm[d] == km[k], d)
            d //= 2
        k *= 2
    pv = pltpu.roll(v, NB - 64, 1)
    pi = pltpu.roll(ix, NB - 64, 1)
    sf = (v > pv) | ((v == pv) & (ix < pi))
    vm = jnp.where(sf, v, pv)
    im = jnp.where(sf, ix, pi)
    vs = pltpu.roll(vm, NB - 64, 1)
    is_ = pltpu.roll(im, NB - 64, 1)
    low = lane < 64
    v2 = jnp.where(low, vm, vs)[:, :128]
    i2 = jnp.where(low, im, is_)[:, :128]
    for d in (32, 16, 8, 4, 2, 1):
        v2, i2 = bstep(v2, i2, 128, bitm2[d], bitm2[d] == km2[64], d)
    for d in (64, 32, 16, 8, 4, 2, 1):
        v2, i2 = bstep(v2, i2, 128, bitm2[d], bitm2[d] == km2[128], d)
    idx_ref[0] = jnp.where(v2[:, :TOPK] > NEG_INF, i2[:, :TOPK], -1)


@jax.jit
def kernel(h, w_a_kv, w_b_kv, w_a_z, w_b_z, bias_a, bias_b, w_dq, w_iuq, w_w):
    del w_a_kv, w_a_z, bias_a

    kic, top_idx = pl.pallas_call(
        _body,
        grid=(B, NT),
        in_specs=[
            pl.BlockSpec((1, TQ, HIDDEN), lambda b, i: (b, i, 0)),
            pl.BlockSpec((C, HIDDEN), lambda b, i: (0, 0)),
            pl.BlockSpec((C, HIDDEN), lambda b, i: (0, 0)),
            pl.BlockSpec((M, C), lambda b, i: (0, 0)),
            pl.BlockSpec((C, HIDDEN), lambda b, i: (0, 0)),
            pl.BlockSpec((NH * C, C), lambda b, i: (0, 0)),
            pl.BlockSpec((NH, HIDDEN), lambda b, i: (0, 0)),
        ],
        out_specs=[
            pl.BlockSpec((1, TQ // M, C), lambda b, i: (b, i, 0)),
            pl.BlockSpec((1, TQ, TOPK), lambda b, i: (b, i, 0)),
        ],
        out_shape=[
            jax.ShapeDtypeStruct((B, NB, C), jnp.float32),
            jax.ShapeDtypeStruct((B, T, TOPK), jnp.int32),
        ],
        scratch_shapes=[pltpu.VMEM((NB, C), jnp.float32)],
    )(h, w_b_kv, w_b_z, bias_b, w_dq, w_iuq, w_w)

    return (kic, top_idx)


# speedup vs baseline: 2.1042x; 1.4969x over previous
import jax
import jax.numpy as jnp
from jax.experimental import pallas as pl
from jax.experimental.pallas import tpu as pltpu

HIDDEN = 2048
C = 64
NH = 16
TOPK = 64
M = 16
B = 2
T = 4096
NB = T // M

TQ = 512
NT = T // TQ
NEG_INF = float("-inf")


def _body(h_ref, w_kv_ref, w_z_ref, bias_ref, w_dq_ref, w_iuq_ref, w_w_ref,
          kic_ref, idx_ref, kic_acc):
    i = pl.program_id(1)
    hb = h_ref[0]
    dn = (((1,), (1,)), ((), ()))

    cb = jax.lax.dot_general(hb, w_kv_ref[...], dn,
                             preferred_element_type=jnp.float32)
    zb = jax.lax.dot_general(hb, w_z_ref[...], dn,
                             preferred_element_type=jnp.float32)
    nblk = TQ // M
    z = zb.reshape(nblk, M, C) + bias_ref[...][None, :, :]
    z = z - jnp.max(z, axis=1, keepdims=True)
    ez = jnp.exp(z)
    wgt = ez / jnp.sum(ez, axis=1, keepdims=True)
    kic = jnp.sum(wgt * cb.reshape(nblk, M, C), axis=1)
    kic_ref[0] = kic
    kic_acc[pl.ds(i * nblk, nblk), :] = kic

    cq = jax.lax.dot_general(hb, w_dq_ref[...], dn,
                             preferred_element_type=jnp.float32)
    q = jax.lax.dot_general(cq, w_iuq_ref[...], dn,
                            preferred_element_type=jnp.float32)
    wv = jax.lax.dot_general(hb, w_w_ref[...], dn,
                             preferred_element_type=jnp.float32)
    kic_all = kic_acc[...]

    scores = jnp.zeros((TQ, NB), jnp.float32)
    for hd in range(NH):
        qh = q[:, hd * C:(hd + 1) * C]
        s = jax.lax.dot_general(qh, kic_all, dn,
                                preferred_element_type=jnp.float32)
        scores = scores + jnp.maximum(s, 0.0) * wv[:, hd:hd + 1]

    tvec = i * TQ + jax.lax.broadcasted_iota(jnp.int32, (TQ, NB), 0)
    bend = jax.lax.broadcasted_iota(jnp.int32, (TQ, NB), 1) * M + (M - 1)
    scores = jnp.where(bend < tvec, scores, NEG_INF)

    lane = jax.lax.broadcasted_iota(jnp.int32, (TQ, NB), 1)
    bitm = {d: (lane & d) != 0 for d in (1, 2, 4, 8, 16, 32, 64, 128)}
    km = {k: (lane & k) == 0 for k in (2, 4, 8, 16, 32, 64, 128)}
    lane2 = lane[:, :128]
    bitm2 = {d: bitm[d][:, :128] for d in (1, 2, 4, 8, 16, 32, 64)}
    km2 = {k: km[k][:, :128] for k in (64, 128)}

    def xor_mat(width, d):
        r = jax.lax.broadcasted_iota(jnp.int32, (width, width), 0)
        c = jax.lax.broadcasted_iota(jnp.int32, (width, width), 1)
        return ((r ^ c) == d).astype(jnp.bfloat16)

    pmat = {d: xor_mat(NB, d) for d in (1, 2, 4, 8, 16, 32)}
    pmat2 = {d: xor_mat(128, d) for d in (1, 2, 4, 8, 16, 32, 64)}
    dnp = (((1,), (0,)), ((), ()))

    def bstep(v, ix, width, bit_d, flip, d, pm):
        pv = jnp.where(bit_d, pltpu.roll(v, d, 1),
                       pltpu.roll(v, width - d, 1))
        pi = jax.lax.dot_general(ix, pm, dnp,
                                 preferred_element_type=jnp.float32
                                 ).astype(jnp.bfloat16)
        sf = (v > pv) | ((v == pv) & (ix < pi))
        take = sf != flip
        return jnp.where(take, v, pv), jnp.where(take, ix, pi)

    v = scores
    ix = lane.astype(jnp.bfloat16)
    k = 2
    while k <= 64:
        d = k // 2
        while d >= 1:
            v, ix = bstep(v, ix, NB, bitm[d], bitm[d] == km[k], d, pmat[d])
            d //= 2
        k *= 2
    pv = pltpu.roll(v, NB - 64, 1)
    pi = pltpu.roll(ix, NB - 64, 1)
    sf = (v > pv) | ((v == pv) & (ix < pi))
    vm = jnp.where(sf, v, pv)
    im = jnp.where(sf, ix, pi)
    vs = pltpu.roll(vm, NB - 64, 1)
    is_ = pltpu.roll(im, NB - 64, 1)
    low = lane < 64
    v2 = jnp.where(low, vm, vs)[:, :128]
    i2 = jnp.where(low, im, is_)[:, :128]
    for d in (32, 16, 8, 4, 2, 1):
        v2, i2 = bstep(v2, i2, 128, bitm2[d], bitm2[d] == km2[64], d,
                       pmat2[d])
    for d in (64, 32, 16, 8, 4, 2, 1):
        v2, i2 = bstep(v2, i2, 128, bitm2[d], bitm2[d] == km2[128], d,
                       pmat2[d])
    idx_ref[0] = jnp.where(v2[:, :TOPK] > NEG_INF,
                           i2[:, :TOPK].astype(jnp.int32), -1)


@jax.jit
def kernel(h, w_a_kv, w_b_kv, w_a_z, w_b_z, bias_a, bias_b, w_dq, w_iuq, w_w):
    del w_a_kv, w_a_z, bias_a

    kic, top_idx = pl.pallas_call(
        _body,
        grid=(B, NT),
        in_specs=[
            pl.BlockSpec((1, TQ, HIDDEN), lambda b, i: (b, i, 0)),
            pl.BlockSpec((C, HIDDEN), lambda b, i: (0, 0)),
            pl.BlockSpec((C, HIDDEN), lambda b, i: (0, 0)),
            pl.BlockSpec((M, C), lambda b, i: (0, 0)),
            pl.BlockSpec((C, HIDDEN), lambda b, i: (0, 0)),
            pl.BlockSpec((NH * C, C), lambda b, i: (0, 0)),
            pl.BlockSpec((NH, HIDDEN), lambda b, i: (0, 0)),
        ],
        out_specs=[
            pl.BlockSpec((1, TQ // M, C), lambda b, i: (b, i, 0)),
            pl.BlockSpec((1, TQ, TOPK), lambda b, i: (b, i, 0)),
        ],
        out_shape=[
            jax.ShapeDtypeStruct((B, NB, C), jnp.float32),
            jax.ShapeDtypeStruct((B, T, TOPK), jnp.int32),
        ],
        scratch_shapes=[pltpu.VMEM((NB, C), jnp.float32)],
    )(h, w_b_kv, w_b_z, bias_b, w_dq, w_iuq, w_w)

    return (kic, top_idx)
